# baseline (speedup 1.0000x reference)
import functools

import jax
import jax.numpy as jnp
from jax import lax
from jax.experimental import pallas as pl
from jax.experimental.pallas import tpu as pltpu
from jax.experimental.pallas import tpu_sc as plsc

N = 1_000_000
EMA_BETA = 0.95
LR_A = 0.01
LR_B = 0.001
K = 100_000

BM = N

NW = 32
CHUNK = 31248
CHUNK_LAST = 31312
NV = CHUNK // 16
NV_EXTRA = (CHUNK_LAST - CHUNK) // 16
CAND = 0x3F000000
NB1, SH1 = 2048, 19
NB2, SH2 = 2048, 8
NB3 = 256

_MESH = plsc.VectorSubcoreMesh(core_axis_name="c", subcore_axis_name="s")


def _mask_body(m_ref, o_ref):
    o_ref[...] = m_ref[...] != 0


def _mask_cast(m):
    return pl.pallas_call(
        _mask_body,
        grid=(1,),
        in_specs=[pl.BlockSpec((BM,), lambda i: (i,))],
        out_specs=pl.BlockSpec((BM,), lambda i: (i,)),
        out_shape=jax.ShapeDtypeStruct((N,), jnp.bool_),
    )(m)


SUB = 1008
NSUB = CHUNK // SUB


def _sqrtv(x):
    b = lax.bitcast_convert_type(x, jnp.int32)
    y = lax.bitcast_convert_type(jnp.int32(0x5F3759DF) - (b >> 1), jnp.float32)
    y = y * (1.5 - 0.5 * x * y * y)
    y = y * (1.5 - 0.5 * x * y * y)
    y = y * (1.5 - 0.5 * x * y * y)
    return x * y


def _wid():
    return lax.axis_index("s") * 2 + lax.axis_index("c")


def _zero_hist(hist, nb):
    z = jnp.zeros((16,), jnp.int32)

    def body(j, _):
        for l in range(16):
            hist[l, pl.ds(j * 16, 16)] = z
        return 0

    lax.fori_loop(0, nb // 16, body, 0)


def _hist_pass(gv, hist, lo, hi, shift, wid):
    lanes = lax.iota(jnp.int32, 16)
    ones = jnp.ones((16,), jnp.int32)

    def one(i):
        bits = gv[pl.ds(i * 16, 16)]
        ok = (bits >= lo) & (bits < hi)
        b = (bits - lo) >> shift
        b = jnp.where(ok, b, 0)
        plsc.addupdate_scatter(hist, [lanes, b], ones, mask=ok)

    def body(i, _):
        one(i)
        return 0

    lax.fori_loop(0, NV, body, 0)

    @pl.when(wid == NW - 1)
    def _():
        for e in range(NV_EXTRA):
            one(NV + e)


def _reduce_hist(hist, outrow, nb):
    def body(j, _):
        s = hist[0, pl.ds(j * 16, 16)]
        for l in range(1, 16):
            s = s + hist[l, pl.ds(j * 16, 16)]
        outrow[pl.ds(j * 16, 16)] = s
        return 0

    lax.fori_loop(0, nb // 16, body, 0)


def _combine(src_hbm, stage, acc, nb):
    for t8 in range(NW // 8):
        pltpu.sync_copy(src_hbm.at[pl.ds(t8 * 8, 8)], stage)

        def body(j, _, first=(t8 == 0)):
            s = stage[0, pl.ds(j * 16, 16)]
            for l in range(1, 8):
                s = s + stage[l, pl.ds(j * 16, 16)]
            if first:
                acc[pl.ds(j * 16, 16)] = s
            else:
                acc[pl.ds(j * 16, 16)] = acc[pl.ds(j * 16, 16)] + s
            return 0

        lax.fori_loop(0, nb // 16, body, 0)


def _find(acc, nb, k):
    lanes = lax.iota(jnp.int32, 16)
    z = jnp.int32(0)

    def body(j, c):
        run, found, binsel, cumb = c
        v = acc[pl.ds(j * 16, 16)]
        cs = plsc.cumsum(v) + run
        cross = cs >= k
        ci = cross.astype(jnp.int32)
        pick = cross & (plsc.cumsum(ci) == 1) & jnp.broadcast_to(found == 0, (16,))
        pi = pick.astype(jnp.int32)
        binsel = binsel + jnp.sum(pi * (j * 16 + lanes))
        cumb = cumb + jnp.sum(pi * (cs - v))
        found = found + jnp.sum(pi)
        run = run + jnp.sum(v)
        return run, found, binsel, cumb

    run, found, binsel, cumb = lax.fori_loop(0, nb // 16, body, (z, z, z, z))
    return found, binsel, cumb


@functools.partial(
    pl.kernel,
    out_type=(
        jax.ShapeDtypeStruct((N,), jnp.int32),
        jax.ShapeDtypeStruct((NW, NB1), jnp.int32),
    ),
    mesh=_MESH,
    compiler_params=pltpu.CompilerParams(use_tc_tiling_on_sc=False, needs_layout_passes=False),
    scratch_types=[
        pltpu.VMEM((SUB, 16), jnp.float32),
        pltpu.VMEM((SUB, 16), jnp.float32),
        pltpu.VMEM((SUB, 1), jnp.float32),
        pltpu.VMEM((CHUNK_LAST,), jnp.int32),
        pltpu.VMEM((16, NB1), jnp.int32),
        pltpu.VMEM((NB1,), jnp.int32),
        pltpu.SemaphoreType.DMA,
        pltpu.SemaphoreType.DMA,
        pltpu.SemaphoreType.DMA,
    ],
)
def _sc_pass1(a_hbm, b_hbm, ge_hbm, g_hbm, h1_hbm,
              av, bv, gev, gv, hist, outrow, sa, sb, sg):
    wid = _wid()
    base = pl.multiple_of(wid * CHUNK, 8)
    lanes = lax.iota(jnp.int32, 16)
    zeros16 = jnp.zeros((16,), jnp.int32)

    def load_sub(row0, nrows):
        ca = pltpu.async_copy(a_hbm.at[pl.ds(row0, nrows)], av.at[pl.ds(0, nrows)], sa)
        cb = pltpu.async_copy(b_hbm.at[pl.ds(row0, nrows)], bv.at[pl.ds(0, nrows)], sb)
        cg = pltpu.async_copy(ge_hbm.at[pl.ds(row0, nrows)], gev.at[pl.ds(0, nrows)], sg)
        ca.wait()
        cb.wait()
        cg.wait()

    def group(goff, lrow):
        ridx = lrow + lanes
        acc_a = jnp.zeros((16,), jnp.float32)
        acc_b = jnp.zeros((16,), jnp.float32)
        for j in range(16):
            jv = jnp.full((16,), j, jnp.int32)
            col_a = plsc.load_gather(av, [ridx, jv])
            acc_a = acc_a + col_a * col_a
            col_b = plsc.load_gather(bv, [ridx, jv])
            acc_b = acc_b + col_b * col_b
        ge = plsc.load_gather(gev, [ridx, zeros16])
        total = LR_A * _sqrtv(acc_a) + LR_B * _sqrtv(acc_b)
        ema = ge * EMA_BETA + (1.0 - EMA_BETA) * total
        gv[pl.ds(goff, 16)] = lax.bitcast_convert_type(ema, jnp.int32)

    def sub_body(s, _):
        load_sub(base + s * SUB, SUB)

        def gbody(gg, _):
            group(s * SUB + gg * 16, gg * 16)
            return 0

        lax.fori_loop(0, SUB // 16, gbody, 0)
        return 0

    lax.fori_loop(0, NSUB, sub_body, 0)

    @pl.when(wid == NW - 1)
    def _():
        load_sub(base + CHUNK, 64)
        for e in range(NV_EXTRA):
            group(CHUNK + e * 16, e * 16)

    _zero_hist(hist, NB1)
    _hist_pass(gv, hist, jnp.int32(0), jnp.int32(CAND), SH1, wid)
    _reduce_hist(hist, outrow, NB1)
    pltpu.sync_copy(outrow, h1_hbm.at[wid])

    @pl.when(wid == NW - 1)
    def _():
        pltpu.sync_copy(gv, g_hbm.at[pl.ds(base, CHUNK_LAST)])

    @pl.when(wid != NW - 1)
    def _():
        pltpu.sync_copy(gv.at[pl.ds(0, CHUNK)], g_hbm.at[pl.ds(base, CHUNK)])


@functools.partial(
    pl.kernel,
    out_type=jax.ShapeDtypeStruct((NW, NB2), jnp.int32),
    mesh=_MESH,
    compiler_params=pltpu.CompilerParams(use_tc_tiling_on_sc=False, needs_layout_passes=False),
    scratch_types=[
        pltpu.VMEM((CHUNK_LAST,), jnp.int32),
        pltpu.VMEM((16, NB2), jnp.int32),
        pltpu.VMEM((NB2,), jnp.int32),
        pltpu.VMEM((8, NB1), jnp.int32),
        pltpu.VMEM((NB1,), jnp.int32),
    ],
)
def _sc_pass2(g_hbm, h1_hbm, h2_hbm, gv, hist, outrow, stage, acc):
    wid = _wid()
    base = pl.multiple_of(wid * CHUNK, 8)
    _combine(h1_hbm, stage, acc, NB1)
    f1, b1, c1 = _find(acc, NB1, jnp.int32(K))
    lo1 = b1 << SH1
    pltpu.sync_copy(g_hbm.at[pl.ds(base, CHUNK_LAST)], gv)
    _zero_hist(hist, NB2)
    _hist_pass(gv, hist, lo1, lo1 + (1 << SH1), SH2, wid)
    _reduce_hist(hist, outrow, NB2)
    pltpu.sync_copy(outrow, h2_hbm.at[wid])


@functools.partial(
    pl.kernel,
    out_type=jax.ShapeDtypeStruct((NW, NB3), jnp.int32),
    mesh=_MESH,
    compiler_params=pltpu.CompilerParams(use_tc_tiling_on_sc=False, needs_layout_passes=False),
    scratch_types=[
        pltpu.VMEM((CHUNK_LAST,), jnp.int32),
        pltpu.VMEM((16, NB3), jnp.int32),
        pltpu.VMEM((NB3,), jnp.int32),
        pltpu.VMEM((8, NB1), jnp.int32),
        pltpu.VMEM((NB1,), jnp.int32),
    ],
)
def _sc_pass3(g_hbm, h1_hbm, h2_hbm, h3_hbm, gv, hist, outrow, stage, acc):
    wid = _wid()
    base = pl.multiple_of(wid * CHUNK, 8)
    _combine(h1_hbm, stage, acc, NB1)
    f1, b1, c1 = _find(acc, NB1, jnp.int32(K))
    lo1 = b1 << SH1
    k1 = K - c1
    _combine(h2_hbm, stage, acc, NB2)
    f2, b2, c2 = _find(acc, NB2, k1)
    lo2 = lo1 + (b2 << SH2)
    pltpu.sync_copy(g_hbm.at[pl.ds(base, CHUNK_LAST)], gv)
    _zero_hist(hist, NB3)
    _hist_pass(gv, hist, lo2, lo2 + NB3, 0, wid)
    _reduce_hist(hist, outrow, NB3)
    pltpu.sync_copy(outrow, h3_hbm.at[wid])


@functools.partial(
    pl.kernel,
    out_type=(
        jax.ShapeDtypeStruct((N,), jnp.int32),
        jax.ShapeDtypeStruct((N,), jnp.int32),
    ),
    mesh=_MESH,
    compiler_params=pltpu.CompilerParams(use_tc_tiling_on_sc=False, needs_layout_passes=False),
    scratch_types=[
        pltpu.VMEM((CHUNK_LAST,), jnp.int32),
        pltpu.VMEM((8, NB1), jnp.int32),
        pltpu.VMEM((NB1,), jnp.int32),
        pltpu.VMEM((NW, NB3), jnp.int32),
        pltpu.VMEM((CHUNK_LAST,), jnp.int32),
        pltpu.VMEM((CHUNK_LAST,), jnp.int32),
    ],
)
def _sc_pass4(g_hbm, h1_hbm, h2_hbm, h3_hbm, pr_hbm, mk_hbm,
              gv, stage, acc, h3s, prv, mkv):
    wid = _wid()
    base = pl.multiple_of(wid * CHUNK, 8)
    _combine(h1_hbm, stage, acc, NB1)
    f1, b1, c1 = _find(acc, NB1, jnp.int32(K))
    lo1 = b1 << SH1
    k1 = K - c1
    _combine(h2_hbm, stage, acc, NB2)
    f2, b2, c2 = _find(acc, NB2, k1)
    lo2 = lo1 + (b2 << SH2)
    k2 = k1 - c2
    pltpu.sync_copy(h3_hbm, h3s)

    def cb3(j, _):
        s = h3s[0, pl.ds(j * 16, 16)]
        for t in range(1, NW):
            s = s + h3s[t, pl.ds(j * 16, 16)]
        acc[pl.ds(j * 16, 16)] = s
        return 0

    lax.fori_loop(0, NB3 // 16, cb3, 0)
    f3, b3, c3 = _find(acc, NB3, k2)
    valid = (f1 + f2 + f3) == 3
    kth = jnp.where(valid, lo2 + b3, jnp.int32(CAND))
    r = jnp.where(valid, k2 - c3, jnp.int32(0))

    lanes = lax.iota(jnp.int32, 16)
    prefix = jnp.int32(0)
    for t in range(NW):
        s = jnp.zeros((16,), jnp.int32)
        for j in range(NB3 // 16):
            s = s + jnp.where(lanes + j * 16 == b3, h3s[t, pl.ds(j * 16, 16)], 0)
        cnt = jnp.sum(s)
        prefix = prefix + jnp.where(wid > t, cnt, 0)

    pltpu.sync_copy(g_hbm.at[pl.ds(base, CHUNK_LAST)], gv)

    def fin(i, tie_run):
        bits = gv[pl.ds(i * 16, 16)]
        below = bits < kth
        tie = bits == kth
        ti = tie.astype(jnp.int32)
        rank = prefix + tie_run + plsc.cumsum(ti) - ti
        prune = below | (tie & (rank < r))
        mkv[pl.ds(i * 16, 16)] = 1 - prune.astype(jnp.int32)
        prv[pl.ds(i * 16, 16)] = jnp.where(prune, 0, bits)
        return tie_run + jnp.sum(ti)

    tr = lax.fori_loop(0, NV, fin, jnp.int32(0))

    @pl.when(wid == NW - 1)
    def _():
        t = tr
        for e in range(NV_EXTRA):
            t = fin(NV + e, t)
        pltpu.sync_copy(prv, pr_hbm.at[pl.ds(base, CHUNK_LAST)])
        pltpu.sync_copy(mkv, mk_hbm.at[pl.ds(base, CHUNK_LAST)])

    @pl.when(wid != NW - 1)
    def _():
        pltpu.sync_copy(prv.at[pl.ds(0, CHUNK)], pr_hbm.at[pl.ds(base, CHUNK)])
        pltpu.sync_copy(mkv.at[pl.ds(0, CHUNK)], mk_hbm.at[pl.ds(base, CHUNK)])


@jax.jit
def kernel(grad_ema, grad_a, grad_b):
    g, h1 = _sc_pass1(grad_a, grad_b, grad_ema)
    h2 = _sc_pass2(g, h1)
    h3 = _sc_pass3(g, h1, h2)
    pruned_bits, mask_i = _sc_pass4(g, h1, h2, h3)
    pruned = lax.bitcast_convert_type(pruned_bits, jnp.float32)
    return pruned.reshape(N, 1), _mask_cast(mask_i)


# speedup vs baseline: 1.7790x; 1.7790x over previous
import functools

import jax
import jax.numpy as jnp
from jax import lax
from jax.experimental import pallas as pl
from jax.experimental.pallas import tpu as pltpu
from jax.experimental.pallas import tpu_sc as plsc

N = 1_000_000
EMA_BETA = 0.95
LR_A = 0.01
LR_B = 0.001
K = 100_000

NR = N // 8
BR = 5000
GRID_A = NR // BR

NW = 32
CHUNK = 31248
CHUNK_LAST = 31312
NV = CHUNK // 16
NV_EXTRA = (CHUNK_LAST - CHUNK) // 16
CAND = 0x3F000000
NB1, SH1 = 2048, 19
NB2, SH2 = 2048, 8
NB3 = 256

_MESH = plsc.VectorSubcoreMesh(core_axis_name="c", subcore_axis_name="s")


def _ema_body(a_ref, b_ref, ge_ref, out_ref):
    rows = lax.broadcasted_iota(jnp.int32, (128, 8), 0)
    cols = lax.broadcasted_iota(jnp.int32, (128, 8), 1)
    m = (rows // 16 == cols).astype(jnp.float32)
    dn = (((1,), (0,)), ((), ()))
    a = a_ref[...]
    b = b_ref[...]
    sa = lax.dot_general(a * a, m, dn, preferred_element_type=jnp.float32)
    sb = lax.dot_general(b * b, m, dn, preferred_element_type=jnp.float32)
    total = LR_A * jnp.sqrt(sa) + LR_B * jnp.sqrt(sb)
    out_ref[...] = ge_ref[...] * EMA_BETA + (1.0 - EMA_BETA) * total


def _ema_stage(a2, b2, ge2):
    return pl.pallas_call(
        _ema_body,
        grid=(GRID_A,),
        in_specs=[
            pl.BlockSpec((BR, 128), lambda i: (i, 0)),
            pl.BlockSpec((BR, 128), lambda i: (i, 0)),
            pl.BlockSpec((BR, 8), lambda i: (i, 0)),
        ],
        out_specs=pl.BlockSpec((BR, 8), lambda i: (i, 0)),
        out_shape=jax.ShapeDtypeStruct((NR, 8), jnp.float32),
    )(a2, b2, ge2)


def _wid():
    return lax.axis_index("s") * 2 + lax.axis_index("c")


def _zero_hist(hist, nb):
    z = jnp.zeros((16,), jnp.int32)

    def body(j, _):
        for l in range(16):
            hist[l, pl.ds(j * 16, 16)] = z
        return 0

    lax.fori_loop(0, nb // 16, body, 0)


def _hist_pass(gv, hist, lo, hi, shift, wid):
    lanes = lax.iota(jnp.int32, 16)
    ones = jnp.ones((16,), jnp.int32)

    def one(i):
        bits = gv[pl.ds(i * 16, 16)]
        ok = (bits >= lo) & (bits < hi)
        b = (bits - lo) >> shift
        b = jnp.where(ok, b, 0)
        plsc.addupdate_scatter(hist, [lanes, b], ones, mask=ok)

    def body(i, _):
        one(i)
        return 0

    lax.fori_loop(0, NV, body, 0)

    @pl.when(wid == NW - 1)
    def _():
        for e in range(NV_EXTRA):
            one(NV + e)


def _reduce_hist(hist, outrow, nb):
    def body(j, _):
        s = hist[0, pl.ds(j * 16, 16)]
        for l in range(1, 16):
            s = s + hist[l, pl.ds(j * 16, 16)]
        outrow[pl.ds(j * 16, 16)] = s
        return 0

    lax.fori_loop(0, nb // 16, body, 0)


def _combine(src_hbm, stage, acc, nb):
    for t8 in range(NW // 8):
        pltpu.sync_copy(src_hbm.at[pl.ds(t8 * 8, 8)], stage)

        def body(j, _, first=(t8 == 0)):
            s = stage[0, pl.ds(j * 16, 16)]
            for l in range(1, 8):
                s = s + stage[l, pl.ds(j * 16, 16)]
            if first:
                acc[pl.ds(j * 16, 16)] = s
            else:
                acc[pl.ds(j * 16, 16)] = acc[pl.ds(j * 16, 16)] + s
            return 0

        lax.fori_loop(0, nb // 16, body, 0)


def _find(acc, nb, k):
    lanes = lax.iota(jnp.int32, 16)
    z = jnp.int32(0)

    def body(j, c):
        run, found, binsel, cumb = c
        v = acc[pl.ds(j * 16, 16)]
        cs = plsc.cumsum(v) + run
        cross = cs >= k
        ci = cross.astype(jnp.int32)
        pick = cross & (plsc.cumsum(ci) == 1) & jnp.broadcast_to(found == 0, (16,))
        pi = pick.astype(jnp.int32)
        binsel = binsel + jnp.sum(pi * (j * 16 + lanes))
        cumb = cumb + jnp.sum(pi * (cs - v))
        found = found + jnp.sum(pi)
        run = run + jnp.sum(v)
        return run, found, binsel, cumb

    run, found, binsel, cumb = lax.fori_loop(0, nb // 16, body, (z, z, z, z))
    return found, binsel, cumb


@functools.partial(
    pl.kernel,
    out_type=jax.ShapeDtypeStruct((NW, NB1), jnp.int32),
    mesh=_MESH,
    compiler_params=pltpu.CompilerParams(use_tc_tiling_on_sc=False, needs_layout_passes=False),
    scratch_types=[
        pltpu.VMEM((CHUNK_LAST,), jnp.int32),
        pltpu.VMEM((16, NB1), jnp.int32),
        pltpu.VMEM((NB1,), jnp.int32),
    ],
)
def _sc_pass1(g_hbm, h1_hbm, gv, hist, outrow):
    wid = _wid()
    base = pl.multiple_of(wid * CHUNK, 8)
    pltpu.sync_copy(g_hbm.at[pl.ds(base, CHUNK_LAST)], gv)
    _zero_hist(hist, NB1)
    _hist_pass(gv, hist, jnp.int32(0), jnp.int32(CAND), SH1, wid)
    _reduce_hist(hist, outrow, NB1)
    pltpu.sync_copy(outrow, h1_hbm.at[wid])


@functools.partial(
    pl.kernel,
    out_type=jax.ShapeDtypeStruct((NW, NB2), jnp.int32),
    mesh=_MESH,
    compiler_params=pltpu.CompilerParams(use_tc_tiling_on_sc=False, needs_layout_passes=False),
    scratch_types=[
        pltpu.VMEM((CHUNK_LAST,), jnp.int32),
        pltpu.VMEM((16, NB2), jnp.int32),
        pltpu.VMEM((NB2,), jnp.int32),
        pltpu.VMEM((8, NB1), jnp.int32),
        pltpu.VMEM((NB1,), jnp.int32),
    ],
)
def _sc_pass2(g_hbm, h1_hbm, h2_hbm, gv, hist, outrow, stage, acc):
    wid = _wid()
    base = pl.multiple_of(wid * CHUNK, 8)
    _combine(h1_hbm, stage, acc, NB1)
    f1, b1, c1 = _find(acc, NB1, jnp.int32(K))
    lo1 = b1 << SH1
    pltpu.sync_copy(g_hbm.at[pl.ds(base, CHUNK_LAST)], gv)
    _zero_hist(hist, NB2)
    _hist_pass(gv, hist, lo1, lo1 + (1 << SH1), SH2, wid)
    _reduce_hist(hist, outrow, NB2)
    pltpu.sync_copy(outrow, h2_hbm.at[wid])


@functools.partial(
    pl.kernel,
    out_type=jax.ShapeDtypeStruct((NW, NB3), jnp.int32),
    mesh=_MESH,
    compiler_params=pltpu.CompilerParams(use_tc_tiling_on_sc=False, needs_layout_passes=False),
    scratch_types=[
        pltpu.VMEM((CHUNK_LAST,), jnp.int32),
        pltpu.VMEM((16, NB3), jnp.int32),
        pltpu.VMEM((NB3,), jnp.int32),
        pltpu.VMEM((8, NB1), jnp.int32),
        pltpu.VMEM((NB1,), jnp.int32),
    ],
)
def _sc_pass3(g_hbm, h1_hbm, h2_hbm, h3_hbm, gv, hist, outrow, stage, acc):
    wid = _wid()
    base = pl.multiple_of(wid * CHUNK, 8)
    _combine(h1_hbm, stage, acc, NB1)
    f1, b1, c1 = _find(acc, NB1, jnp.int32(K))
    lo1 = b1 << SH1
    k1 = K - c1
    _combine(h2_hbm, stage, acc, NB2)
    f2, b2, c2 = _find(acc, NB2, k1)
    lo2 = lo1 + (b2 << SH2)
    pltpu.sync_copy(g_hbm.at[pl.ds(base, CHUNK_LAST)], gv)
    _zero_hist(hist, NB3)
    _hist_pass(gv, hist, lo2, lo2 + NB3, 0, wid)
    _reduce_hist(hist, outrow, NB3)
    pltpu.sync_copy(outrow, h3_hbm.at[wid])


@functools.partial(
    pl.kernel,
    out_type=(
        jax.ShapeDtypeStruct((N,), jnp.int32),
        jax.ShapeDtypeStruct((N,), jnp.int32),
    ),
    mesh=_MESH,
    compiler_params=pltpu.CompilerParams(use_tc_tiling_on_sc=False, needs_layout_passes=False),
    scratch_types=[
        pltpu.VMEM((CHUNK_LAST,), jnp.int32),
        pltpu.VMEM((8, NB1), jnp.int32),
        pltpu.VMEM((NB1,), jnp.int32),
        pltpu.VMEM((NW, NB3), jnp.int32),
        pltpu.VMEM((CHUNK_LAST,), jnp.int32),
        pltpu.VMEM((CHUNK_LAST,), jnp.int32),
    ],
)
def _sc_pass4(g_hbm, h1_hbm, h2_hbm, h3_hbm, pr_hbm, mk_hbm,
              gv, stage, acc, h3s, prv, mkv):
    wid = _wid()
    base = pl.multiple_of(wid * CHUNK, 8)
    _combine(h1_hbm, stage, acc, NB1)
    f1, b1, c1 = _find(acc, NB1, jnp.int32(K))
    lo1 = b1 << SH1
    k1 = K - c1
    _combine(h2_hbm, stage, acc, NB2)
    f2, b2, c2 = _find(acc, NB2, k1)
    lo2 = lo1 + (b2 << SH2)
    k2 = k1 - c2
    pltpu.sync_copy(h3_hbm, h3s)

    def cb3(j, _):
        s = h3s[0, pl.ds(j * 16, 16)]
        for t in range(1, NW):
            s = s + h3s[t, pl.ds(j * 16, 16)]
        acc[pl.ds(j * 16, 16)] = s
        return 0

    lax.fori_loop(0, NB3 // 16, cb3, 0)
    f3, b3, c3 = _find(acc, NB3, k2)
    valid = (f1 + f2 + f3) == 3
    kth = jnp.where(valid, lo2 + b3, jnp.int32(CAND))
    r = jnp.where(valid, k2 - c3, jnp.int32(0))

    lanes = lax.iota(jnp.int32, 16)
    prefix = jnp.int32(0)
    for t in range(NW):
        s = jnp.zeros((16,), jnp.int32)
        for j in range(NB3 // 16):
            s = s + jnp.where(lanes + j * 16 == b3, h3s[t, pl.ds(j * 16, 16)], 0)
        cnt = jnp.sum(s)
        prefix = prefix + jnp.where(wid > t, cnt, 0)

    pltpu.sync_copy(g_hbm.at[pl.ds(base, CHUNK_LAST)], gv)

    def fin(i, tie_run):
        bits = gv[pl.ds(i * 16, 16)]
        below = bits < kth
        tie = bits == kth
        ti = tie.astype(jnp.int32)
        rank = prefix + tie_run + plsc.cumsum(ti) - ti
        prune = below | (tie & (rank < r))
        mkv[pl.ds(i * 16, 16)] = 1 - prune.astype(jnp.int32)
        prv[pl.ds(i * 16, 16)] = jnp.where(prune, 0, bits)
        return tie_run + jnp.sum(ti)

    tr = lax.fori_loop(0, NV, fin, jnp.int32(0))

    @pl.when(wid == NW - 1)
    def _():
        t = tr
        for e in range(NV_EXTRA):
            t = fin(NV + e, t)
        pltpu.sync_copy(prv, pr_hbm.at[pl.ds(base, CHUNK_LAST)])
        pltpu.sync_copy(mkv, mk_hbm.at[pl.ds(base, CHUNK_LAST)])

    @pl.when(wid != NW - 1)
    def _():
        pltpu.sync_copy(prv.at[pl.ds(0, CHUNK)], pr_hbm.at[pl.ds(base, CHUNK)])
        pltpu.sync_copy(mkv.at[pl.ds(0, CHUNK)], mk_hbm.at[pl.ds(base, CHUNK)])


@jax.jit
def kernel(grad_ema, grad_a, grad_b):
    a2 = grad_a.reshape(NR, 128)
    b2 = grad_b.reshape(NR, 128)
    ge2 = grad_ema.reshape(NR, 8)
    ema2 = _ema_stage(a2, b2, ge2)
    g = lax.bitcast_convert_type(ema2.reshape(N), jnp.int32)
    h1 = _sc_pass1(g)
    h2 = _sc_pass2(g, h1)
    h3 = _sc_pass3(g, h1, h2)
    pruned_bits, mask_i = _sc_pass4(g, h1, h2, h3)
    pruned = lax.bitcast_convert_type(pruned_bits, jnp.float32)
    return pruned.reshape(N, 1), mask_i.astype(jnp.bool_)
